# baseline (speedup 1.0000x reference)
import functools
import math

import jax
import jax.numpy as jnp
from jax.experimental import pallas as pl

N_EMBD = 768
N_HEAD = 12
HEAD_DIM = N_EMBD // N_HEAD
T = 2048
SCALE = 1.0 / math.sqrt(HEAD_DIM)
STRIDES = (1, 3, 7, 13)

BQ = 256
BR = 256
BK = 128


def _proj2_body(x_ref, wqk_ref, bqk_ref, wv_ref, bv_ref, sel_ref,
                qk_ref, v_ref, kmax_ref):
    xb = x_ref[:]
    qkb = jax.lax.dot_general(
        xb, wqk_ref[:], (((1,), (1,)), ((), ())),
        preferred_element_type=jnp.float32) + bqk_ref[:]
    qk_ref[:] = qkb
    v_ref[:] = jax.lax.dot_general(
        xb, wv_ref[:], (((1,), (1,)), ((), ())),
        preferred_element_type=jnp.float32) + bv_ref[:]

    n2 = jax.lax.dot_general(
        qkb * qkb, sel_ref[:], (((1,), (0,)), ((), ())),
        preferred_element_type=jnp.float32)
    blockmax = jnp.max(n2, axis=0, keepdims=True)

    @pl.when(pl.program_id(0) == 0)
    def _init():
        kmax_ref[:] = blockmax

    @pl.when(pl.program_id(0) != 0)
    def _acc():
        kmax_ref[:] = jnp.maximum(kmax_ref[:], blockmax)


def _attn_body(qk_ref, v_ref, kmax_ref, wout_ref, bout_ref, o_ref):
    qi = pl.program_id(0)
    q_all = qk_ref[pl.ds(qi * BQ, BQ), :]
    kmax = kmax_ref[:]

    qh = [q_all[:, h * HEAD_DIM:(h + 1) * HEAD_DIM].astype(jnp.bfloat16)
          for h in range(N_HEAD)]
    bound = [SCALE * kmax[:, h:h + 1] for h in range(N_HEAD)]

    i = qi * BQ + jax.lax.broadcasted_iota(jnp.int32, (BQ, BK), 0)
    p_loc = jax.lax.broadcasted_iota(jnp.int32, (BQ, BK), 1)

    def body(kb, carry):
        ls, accs = carry
        k_blk = qk_ref[pl.ds(kb * BK, BK), :].astype(jnp.bfloat16)
        v_blk = v_ref[pl.ds(kb * BK, BK), :]

        p = kb * BK + p_loc
        d = (p - i) & (T - 1)
        causal = p <= i
        half = d < (T // 2)
        masks = []
        for s in STRIDES:
            if s == 1:
                masks.append(causal)
            else:
                rA = (-(T // 2)) % s
                rB = (T // 2) % s
                ds = d % s
                masks.append(
                    causal & ((half & (ds == rA)) | (~half & (ds == rB))))

        new_ls, new_accs = [], []
        for h in range(N_HEAD):
            sl = slice(h * HEAD_DIM, (h + 1) * HEAD_DIM)
            logits = jax.lax.dot_general(
                qh[h], k_blk[:, sl], (((1,), (1,)), ((), ())),
                preferred_element_type=jnp.float32) * SCALE
            e = jnp.where(masks[h % 4], jnp.exp(logits - bound[h]), 0.0)
            new_ls.append(ls[h] + jnp.sum(e, axis=1, keepdims=True))
            new_accs.append(accs[h] + jax.lax.dot_general(
                e.astype(jnp.bfloat16), v_blk[:, sl].astype(jnp.bfloat16),
                (((1,), (0,)), ((), ())),
                preferred_element_type=jnp.float32))
        return tuple(new_ls), tuple(new_accs)

    ls0 = tuple(jnp.zeros((BQ, 1), jnp.float32) for _ in range(N_HEAD))
    accs0 = tuple(jnp.zeros((BQ, HEAD_DIM), jnp.float32) for _ in range(N_HEAD))
    n_kb = qi * BQ // BK + 1
    ls, accs = jax.lax.fori_loop(0, n_kb, body, (ls0, accs0))

    outs = [jnp.where(ls[h] > 0, accs[h] / jnp.maximum(ls[h], 1e-30), 0.0)
            for h in range(N_HEAD)]
    attn = jnp.concatenate(outs, axis=1)
    o_ref[:] = jax.lax.dot_general(
        attn, wout_ref[:], (((1,), (1,)), ((), ())),
        preferred_element_type=jnp.float32) + bout_ref[:]


@functools.partial(jax.jit, static_argnames=("interpret",))
def _run(x, W_qk, b_qk, W_v, b_v, W_out, b_out, interpret=False):
    x2 = x.reshape(T, N_EMBD)
    nr = T // BR
    sel = jnp.repeat(jnp.eye(N_HEAD, dtype=jnp.float32), HEAD_DIM, axis=0)

    qk, vv, kmax = pl.pallas_call(
        _proj2_body,
        grid=(nr,),
        in_specs=[
            pl.BlockSpec((BR, N_EMBD), lambda r: (r, 0)),
            pl.BlockSpec((N_EMBD, N_EMBD), lambda r: (0, 0)),
            pl.BlockSpec((N_EMBD,), lambda r: (0,)),
            pl.BlockSpec((N_EMBD, N_EMBD), lambda r: (0, 0)),
            pl.BlockSpec((N_EMBD,), lambda r: (0,)),
            pl.BlockSpec((N_EMBD, N_HEAD), lambda r: (0, 0)),
        ],
        out_specs=[
            pl.BlockSpec((BR, N_EMBD), lambda r: (r, 0)),
            pl.BlockSpec((BR, N_EMBD), lambda r: (r, 0)),
            pl.BlockSpec((1, N_HEAD), lambda r: (0, 0)),
        ],
        out_shape=[
            jax.ShapeDtypeStruct((T, N_EMBD), jnp.float32),
            jax.ShapeDtypeStruct((T, N_EMBD), jnp.float32),
            jax.ShapeDtypeStruct((1, N_HEAD), jnp.float32),
        ],
        interpret=interpret,
    )(x2, W_qk, b_qk, W_v, b_v, sel)

    nq = T // BQ
    out = pl.pallas_call(
        _attn_body,
        grid=(nq,),
        in_specs=[
            pl.BlockSpec((T, N_EMBD), lambda qi: (0, 0)),
            pl.BlockSpec((T, N_EMBD), lambda qi: (0, 0)),
            pl.BlockSpec((1, N_HEAD), lambda qi: (0, 0)),
            pl.BlockSpec((N_EMBD, N_EMBD), lambda qi: (0, 0)),
            pl.BlockSpec((N_EMBD,), lambda qi: (0,)),
        ],
        out_specs=pl.BlockSpec((BQ, N_EMBD), lambda qi: (qi, 0)),
        out_shape=jax.ShapeDtypeStruct((T, N_EMBD), jnp.float32),
        interpret=interpret,
    )(qk, vv, kmax, W_out, b_out)

    return out.reshape(1, T, N_EMBD)


def kernel(x, W_qk, b_qk, W_v, b_v, W_out, b_out):
    return _run(x, W_qk, b_qk, W_v, b_v, W_out, b_out)


# speedup vs baseline: 1.3578x; 1.3578x over previous
import functools
import math

import jax
import jax.numpy as jnp
from jax.experimental import pallas as pl

N_EMBD = 768
N_HEAD = 12
HEAD_DIM = N_EMBD // N_HEAD
T = 2048
SCALE = 1.0 / math.sqrt(HEAD_DIM)
STRIDES = (1, 3, 7, 13)

BQ = 256
BR = 256
BK = 256
DV = HEAD_DIM + 1


def _proj2_body(x_ref, wqk_ref, bqk_ref, wv_ref, bv_ref, sel_ref,
                qk_ref, v_ref, kmax_ref):
    xb = x_ref[:]
    qkb = jax.lax.dot_general(
        xb, wqk_ref[:], (((1,), (1,)), ((), ())),
        preferred_element_type=jnp.float32) + bqk_ref[:]
    qk_ref[:] = qkb.astype(jnp.bfloat16)
    vb = jax.lax.dot_general(
        xb, wv_ref[:], (((1,), (1,)), ((), ())),
        preferred_element_type=jnp.float32) + bv_ref[:]
    ones = jnp.ones((vb.shape[0], 1), jnp.bfloat16)
    vcols = []
    for h in range(N_HEAD):
        vcols.append(vb[:, h * HEAD_DIM:(h + 1) * HEAD_DIM].astype(jnp.bfloat16))
        vcols.append(ones)
    v_ref[:] = jnp.concatenate(vcols, axis=1)

    n2 = jax.lax.dot_general(
        qkb * qkb, sel_ref[:], (((1,), (0,)), ((), ())),
        preferred_element_type=jnp.float32)
    blockmax = jnp.max(n2, axis=0, keepdims=True)

    @pl.when(pl.program_id(0) == 0)
    def _init():
        kmax_ref[:] = blockmax

    @pl.when(pl.program_id(0) != 0)
    def _acc():
        kmax_ref[:] = jnp.maximum(kmax_ref[:], blockmax)


def _attn_body(qk_ref, v_ref, kmax_ref, wout_ref, bout_ref, o_ref):
    qi = pl.program_id(0)
    q_all = qk_ref[pl.ds(qi * BQ, BQ), :]
    kmax = kmax_ref[:]

    qh = [q_all[:, h * HEAD_DIM:(h + 1) * HEAD_DIM] for h in range(N_HEAD)]
    bound = [SCALE * kmax[:, h:h + 1] for h in range(N_HEAD)]

    i = qi * BQ + jax.lax.broadcasted_iota(jnp.int32, (BQ, BK), 0)
    p_loc = jax.lax.broadcasted_iota(jnp.int32, (BQ, BK), 1)

    def body(kb, accs):
        k_blk = qk_ref[pl.ds(kb * BK, BK), :]
        v_blk = v_ref[pl.ds(kb * BK, BK), :]

        p = kb * BK + p_loc
        d = (p - i) & (T - 1)
        causal = p <= i
        half = d < (T // 2)
        masks = []
        for s in STRIDES:
            if s == 1:
                masks.append(causal)
            else:
                rA = (-(T // 2)) % s
                rB = (T // 2) % s
                ds = d % s
                masks.append(
                    causal & ((half & (ds == rA)) | (~half & (ds == rB))))

        new_accs = []
        for h in range(N_HEAD):
            logits = jax.lax.dot_general(
                qh[h], k_blk[:, h * HEAD_DIM:(h + 1) * HEAD_DIM],
                (((1,), (1,)), ((), ())),
                preferred_element_type=jnp.float32) * SCALE
            e = jnp.where(masks[h % 4], jnp.exp(logits - bound[h]), 0.0)
            new_accs.append(accs[h] + jax.lax.dot_general(
                e.astype(jnp.bfloat16), v_blk[:, h * DV:(h + 1) * DV],
                (((1,), (0,)), ((), ())),
                preferred_element_type=jnp.float32))
        return tuple(new_accs)

    accs0 = tuple(jnp.zeros((BQ, DV), jnp.float32) for _ in range(N_HEAD))
    n_kb = ((qi + 1) * BQ + BK - 1) // BK
    accs = jax.lax.fori_loop(0, n_kb, body, accs0)

    outs = []
    for h in range(N_HEAD):
        l = accs[h][:, HEAD_DIM:HEAD_DIM + 1]
        outs.append(jnp.where(l > 0,
                              accs[h][:, :HEAD_DIM] / jnp.maximum(l, 1e-30),
                              0.0))
    attn = jnp.concatenate(outs, axis=1).astype(jnp.bfloat16)
    o_ref[:] = jax.lax.dot_general(
        attn, wout_ref[:], (((1,), (1,)), ((), ())),
        preferred_element_type=jnp.float32) + bout_ref[:]


@functools.partial(jax.jit, static_argnames=("interpret",))
def _run(x, W_qk, b_qk, W_v, b_v, W_out, b_out, interpret=False):
    x2 = x.reshape(T, N_EMBD)
    nr = T // BR
    sel = jnp.repeat(jnp.eye(N_HEAD, dtype=jnp.float32), HEAD_DIM, axis=0)

    qk, vv, kmax = pl.pallas_call(
        _proj2_body,
        grid=(nr,),
        in_specs=[
            pl.BlockSpec((BR, N_EMBD), lambda r: (r, 0)),
            pl.BlockSpec((N_EMBD, N_EMBD), lambda r: (0, 0)),
            pl.BlockSpec((N_EMBD,), lambda r: (0,)),
            pl.BlockSpec((N_EMBD, N_EMBD), lambda r: (0, 0)),
            pl.BlockSpec((N_EMBD,), lambda r: (0,)),
            pl.BlockSpec((N_EMBD, N_HEAD), lambda r: (0, 0)),
        ],
        out_specs=[
            pl.BlockSpec((BR, N_EMBD), lambda r: (r, 0)),
            pl.BlockSpec((BR, N_HEAD * DV), lambda r: (r, 0)),
            pl.BlockSpec((1, N_HEAD), lambda r: (0, 0)),
        ],
        out_shape=[
            jax.ShapeDtypeStruct((T, N_EMBD), jnp.bfloat16),
            jax.ShapeDtypeStruct((T, N_HEAD * DV), jnp.bfloat16),
            jax.ShapeDtypeStruct((1, N_HEAD), jnp.float32),
        ],
        interpret=interpret,
    )(x2, W_qk, b_qk, W_v, b_v, sel)

    nq = T // BQ
    out = pl.pallas_call(
        _attn_body,
        grid=(nq,),
        in_specs=[
            pl.BlockSpec((T, N_EMBD), lambda qi: (0, 0)),
            pl.BlockSpec((T, N_HEAD * DV), lambda qi: (0, 0)),
            pl.BlockSpec((1, N_HEAD), lambda qi: (0, 0)),
            pl.BlockSpec((N_EMBD, N_EMBD), lambda qi: (0, 0)),
            pl.BlockSpec((N_EMBD,), lambda qi: (0,)),
        ],
        out_specs=pl.BlockSpec((BQ, N_EMBD), lambda qi: (qi, 0)),
        out_shape=jax.ShapeDtypeStruct((T, N_EMBD), jnp.float32),
        interpret=interpret,
    )(qk, vv, kmax, W_out.astype(jnp.bfloat16), b_out)

    return out.reshape(1, T, N_EMBD)


def kernel(x, W_qk, b_qk, W_v, b_v, W_out, b_out):
    return _run(x, W_qk, b_qk, W_v, b_v, W_out, b_out)


# speedup vs baseline: 1.6109x; 1.1864x over previous
import functools
import math

import jax
import jax.numpy as jnp
import numpy as np
from jax.experimental import pallas as pl

N_EMBD = 768
N_HEAD = 12
HEAD_DIM = N_EMBD // N_HEAD
T = 2048
SCALE = 1.0 / math.sqrt(HEAD_DIM)
STRIDES = (1, 3, 7, 13)

BQ = 256
BR = 256
BK = 256
DV = HEAD_DIM + 1


def _mask_tiles():
    ii = np.arange(BQ)[:, None]
    jj = np.arange(BK)[None, :]
    tiles = np.zeros((len(STRIDES), T // BK, BQ, BK), np.float32)
    for si, s in enumerate(STRIDES):
        for delta in range(T // BK):
            i = delta * BQ + ii
            d = (jj - i) % T
            causal = jj <= i
            if s == 1:
                valid = causal
            else:
                rA = (-(T // 2)) % s
                rB = (T // 2) % s
                valid = causal & np.where(d < T // 2, d % s == rA, d % s == rB)
            tiles[si, delta] = valid
    return tiles


_MASK_TILES = _mask_tiles()


def _proj2_body(x_ref, wqk_ref, bqk_ref, wv_ref, bv_ref, sel_ref,
                qk_ref, v_ref, kmax_ref):
    xb = x_ref[:]
    qkb = jax.lax.dot_general(
        xb, wqk_ref[:], (((1,), (1,)), ((), ())),
        preferred_element_type=jnp.float32) + bqk_ref[:]
    qk_ref[:] = qkb.astype(jnp.bfloat16)
    vb = jax.lax.dot_general(
        xb, wv_ref[:], (((1,), (1,)), ((), ())),
        preferred_element_type=jnp.float32) + bv_ref[:]
    ones = jnp.ones((vb.shape[0], 1), jnp.bfloat16)
    vcols = []
    for h in range(N_HEAD):
        vcols.append(vb[:, h * HEAD_DIM:(h + 1) * HEAD_DIM].astype(jnp.bfloat16))
        vcols.append(ones)
    v_ref[:] = jnp.concatenate(vcols, axis=1)

    n2 = jax.lax.dot_general(
        qkb * qkb, sel_ref[:], (((1,), (0,)), ((), ())),
        preferred_element_type=jnp.float32)
    blockmax = jnp.max(n2, axis=0, keepdims=True)

    @pl.when(pl.program_id(0) == 0)
    def _init():
        kmax_ref[:] = blockmax

    @pl.when(pl.program_id(0) != 0)
    def _acc():
        kmax_ref[:] = jnp.maximum(kmax_ref[:], blockmax)


def _attn_body(qk_ref, v_ref, kmax_ref, tiles_ref, wout_ref, bout_ref, o_ref):
    qi = pl.program_id(0)
    q_all = qk_ref[pl.ds(qi * BQ, BQ), :]
    kmax = kmax_ref[:]

    qh = [q_all[:, h * HEAD_DIM:(h + 1) * HEAD_DIM] * jnp.bfloat16(SCALE)
          for h in range(N_HEAD)]
    bound = [SCALE * kmax[:, h:h + 1] for h in range(N_HEAD)]

    def body(kb, accs):
        k_blk = qk_ref[pl.ds(kb * BK, BK), :]
        v_blk = v_ref[pl.ds(kb * BK, BK), :]
        delta = qi - kb
        tiles = [tiles_ref[si, delta] for si in range(len(STRIDES))]

        new_accs = []
        for h in range(N_HEAD):
            logits = jax.lax.dot_general(
                qh[h], k_blk[:, h * HEAD_DIM:(h + 1) * HEAD_DIM],
                (((1,), (1,)), ((), ())),
                preferred_element_type=jnp.float32)
            e = jnp.exp(logits - bound[h]) * tiles[h % 4]
            new_accs.append(accs[h] + jax.lax.dot_general(
                e.astype(jnp.bfloat16), v_blk[:, h * DV:(h + 1) * DV],
                (((1,), (0,)), ((), ())),
                preferred_element_type=jnp.float32))
        return tuple(new_accs)

    accs0 = tuple(jnp.zeros((BQ, DV), jnp.float32) for _ in range(N_HEAD))
    n_kb = ((qi + 1) * BQ + BK - 1) // BK
    accs = jax.lax.fori_loop(0, n_kb, body, accs0)

    outs = []
    for h in range(N_HEAD):
        l = accs[h][:, HEAD_DIM:HEAD_DIM + 1]
        outs.append(jnp.where(l > 0,
                              accs[h][:, :HEAD_DIM] / jnp.maximum(l, 1e-30),
                              0.0))
    attn = jnp.concatenate(outs, axis=1).astype(jnp.bfloat16)
    o_ref[:] = jax.lax.dot_general(
        attn, wout_ref[:], (((1,), (1,)), ((), ())),
        preferred_element_type=jnp.float32) + bout_ref[:]


@functools.partial(jax.jit, static_argnames=("interpret",))
def _run(x, W_qk, b_qk, W_v, b_v, W_out, b_out, interpret=False):
    x2 = x.reshape(T, N_EMBD)
    nr = T // BR
    sel = jnp.repeat(jnp.eye(N_HEAD, dtype=jnp.float32), HEAD_DIM, axis=0)

    qk, vv, kmax = pl.pallas_call(
        _proj2_body,
        grid=(nr,),
        in_specs=[
            pl.BlockSpec((BR, N_EMBD), lambda r: (r, 0)),
            pl.BlockSpec((N_EMBD, N_EMBD), lambda r: (0, 0)),
            pl.BlockSpec((N_EMBD,), lambda r: (0,)),
            pl.BlockSpec((N_EMBD, N_EMBD), lambda r: (0, 0)),
            pl.BlockSpec((N_EMBD,), lambda r: (0,)),
            pl.BlockSpec((N_EMBD, N_HEAD), lambda r: (0, 0)),
        ],
        out_specs=[
            pl.BlockSpec((BR, N_EMBD), lambda r: (r, 0)),
            pl.BlockSpec((BR, N_HEAD * DV), lambda r: (r, 0)),
            pl.BlockSpec((1, N_HEAD), lambda r: (0, 0)),
        ],
        out_shape=[
            jax.ShapeDtypeStruct((T, N_EMBD), jnp.bfloat16),
            jax.ShapeDtypeStruct((T, N_HEAD * DV), jnp.bfloat16),
            jax.ShapeDtypeStruct((1, N_HEAD), jnp.float32),
        ],
        interpret=interpret,
    )(x2, W_qk, b_qk, W_v, b_v, sel)

    nq = T // BQ
    tiles = jnp.asarray(_MASK_TILES)
    out = pl.pallas_call(
        _attn_body,
        grid=(nq,),
        in_specs=[
            pl.BlockSpec((T, N_EMBD), lambda qi: (0, 0)),
            pl.BlockSpec((T, N_HEAD * DV), lambda qi: (0, 0)),
            pl.BlockSpec((1, N_HEAD), lambda qi: (0, 0)),
            pl.BlockSpec((len(STRIDES), T // BK, BQ, BK), lambda qi: (0, 0, 0, 0)),
            pl.BlockSpec((N_EMBD, N_EMBD), lambda qi: (0, 0)),
            pl.BlockSpec((N_EMBD,), lambda qi: (0,)),
        ],
        out_specs=pl.BlockSpec((BQ, N_EMBD), lambda qi: (qi, 0)),
        out_shape=jax.ShapeDtypeStruct((T, N_EMBD), jnp.float32),
        interpret=interpret,
    )(qk, vv, kmax, tiles, W_out.astype(jnp.bfloat16), b_out)

    return out.reshape(1, T, N_EMBD)


def kernel(x, W_qk, b_qk, W_v, b_v, W_out, b_out):
    return _run(x, W_qk, b_qk, W_v, b_v, W_out, b_out)


# speedup vs baseline: 1.6413x; 1.0188x over previous
import functools
import math

import jax
import jax.numpy as jnp
import numpy as np
from jax.experimental import pallas as pl

N_EMBD = 768
N_HEAD = 12
HEAD_DIM = N_EMBD // N_HEAD
T = 2048
SCALE = 1.0 / math.sqrt(HEAD_DIM)
STRIDES = (1, 3, 7, 13)

BQ = 256
BR = 256
BK = 256
DV = HEAD_DIM + 1


def _mask_tiles():
    ii = np.arange(BQ)[:, None]
    jj = np.arange(BK)[None, :]
    tiles = np.zeros((len(STRIDES), T // BK, BQ, BK), np.float32)
    for si, s in enumerate(STRIDES):
        for delta in range(T // BK):
            i = delta * BQ + ii
            d = (jj - i) % T
            causal = jj <= i
            if s == 1:
                valid = causal
            else:
                rA = (-(T // 2)) % s
                rB = (T // 2) % s
                valid = causal & np.where(d < T // 2, d % s == rA, d % s == rB)
            tiles[si, delta] = valid
    return tiles


_MASK_TILES = _mask_tiles()


def _proj2_body(x_ref, wqk_ref, bqk_ref, wv_ref, bv_ref, sel_ref,
                qk_ref, v_ref, kmax_ref):
    xb = x_ref[:]
    qkb = jax.lax.dot_general(
        xb, wqk_ref[:], (((1,), (1,)), ((), ())),
        preferred_element_type=jnp.float32) + bqk_ref[:]
    qk_ref[:] = qkb.astype(jnp.bfloat16)
    vb = jax.lax.dot_general(
        xb, wv_ref[:], (((1,), (1,)), ((), ())),
        preferred_element_type=jnp.float32) + bv_ref[:]
    ones = jnp.ones((vb.shape[0], 1), jnp.bfloat16)
    vcols = []
    for h in range(N_HEAD):
        vcols.append(vb[:, h * HEAD_DIM:(h + 1) * HEAD_DIM].astype(jnp.bfloat16))
        vcols.append(ones)
    v_ref[:] = jnp.concatenate(vcols, axis=1)

    n2 = jax.lax.dot_general(
        qkb * qkb, sel_ref[:], (((1,), (0,)), ((), ())),
        preferred_element_type=jnp.float32)
    blockmax = jnp.max(n2, axis=0, keepdims=True)

    @pl.when(pl.program_id(0) == 0)
    def _init():
        kmax_ref[:] = blockmax

    @pl.when(pl.program_id(0) != 0)
    def _acc():
        kmax_ref[:] = jnp.maximum(kmax_ref[:], blockmax)


def _attn_body(qk_ref, v_ref, kmax_ref, tiles_ref, wout_ref, bout_ref, o_ref):
    qi = pl.program_id(0)
    q_all = qk_ref[pl.ds(qi * BQ, BQ), :]
    kmax = kmax_ref[:]

    qh = [q_all[:, h * HEAD_DIM:(h + 1) * HEAD_DIM] * jnp.bfloat16(SCALE)
          for h in range(N_HEAD)]
    bound = [SCALE * kmax[:, h:h + 1] for h in range(N_HEAD)]

    def body(kb, accs):
        k_blk = qk_ref[pl.ds(kb * BK, BK), :]
        v_blk = v_ref[pl.ds(kb * BK, BK), :]
        delta = qi - kb
        tiles = [tiles_ref[si, delta] for si in range(len(STRIDES))]

        new_accs = []
        for h in range(N_HEAD):
            logits = jax.lax.dot_general(
                qh[h], k_blk[:, h * HEAD_DIM:(h + 1) * HEAD_DIM],
                (((1,), (1,)), ((), ())),
                preferred_element_type=jnp.float32)
            e16 = jnp.exp(logits - bound[h]).astype(jnp.bfloat16) * tiles[h % 4]
            new_accs.append(accs[h] + jax.lax.dot_general(
                e16, v_blk[:, h * DV:(h + 1) * DV],
                (((1,), (0,)), ((), ())),
                preferred_element_type=jnp.float32))
        return tuple(new_accs)

    accs0 = tuple(jnp.zeros((BQ, DV), jnp.float32) for _ in range(N_HEAD))
    n_kb = ((qi + 1) * BQ + BK - 1) // BK
    accs = jax.lax.fori_loop(0, n_kb, body, accs0)

    outs = []
    for h in range(N_HEAD):
        l = accs[h][:, HEAD_DIM:HEAD_DIM + 1]
        outs.append(jnp.where(l > 0,
                              accs[h][:, :HEAD_DIM] / jnp.maximum(l, 1e-30),
                              0.0))
    attn = jnp.concatenate(outs, axis=1).astype(jnp.bfloat16)
    o_ref[:] = jax.lax.dot_general(
        attn, wout_ref[:], (((1,), (1,)), ((), ())),
        preferred_element_type=jnp.float32) + bout_ref[:]


@functools.partial(jax.jit, static_argnames=("interpret",))
def _run(x, W_qk, b_qk, W_v, b_v, W_out, b_out, interpret=False):
    x2 = x.reshape(T, N_EMBD)
    nr = T // BR
    sel = jnp.repeat(jnp.eye(N_HEAD, dtype=jnp.float32), HEAD_DIM, axis=0)

    qk, vv, kmax = pl.pallas_call(
        _proj2_body,
        grid=(nr,),
        in_specs=[
            pl.BlockSpec((BR, N_EMBD), lambda r: (r, 0)),
            pl.BlockSpec((N_EMBD, N_EMBD), lambda r: (0, 0)),
            pl.BlockSpec((N_EMBD,), lambda r: (0,)),
            pl.BlockSpec((N_EMBD, N_EMBD), lambda r: (0, 0)),
            pl.BlockSpec((N_EMBD,), lambda r: (0,)),
            pl.BlockSpec((N_EMBD, N_HEAD), lambda r: (0, 0)),
        ],
        out_specs=[
            pl.BlockSpec((BR, N_EMBD), lambda r: (r, 0)),
            pl.BlockSpec((BR, N_HEAD * DV), lambda r: (r, 0)),
            pl.BlockSpec((1, N_HEAD), lambda r: (0, 0)),
        ],
        out_shape=[
            jax.ShapeDtypeStruct((T, N_EMBD), jnp.bfloat16),
            jax.ShapeDtypeStruct((T, N_HEAD * DV), jnp.bfloat16),
            jax.ShapeDtypeStruct((1, N_HEAD), jnp.float32),
        ],
        interpret=interpret,
    )(x2, W_qk, b_qk, W_v, b_v, sel)

    nq = T // BQ
    tiles = jnp.asarray(_MASK_TILES).astype(jnp.bfloat16)
    out = pl.pallas_call(
        _attn_body,
        grid=(nq,),
        in_specs=[
            pl.BlockSpec((T, N_EMBD), lambda qi: (0, 0)),
            pl.BlockSpec((T, N_HEAD * DV), lambda qi: (0, 0)),
            pl.BlockSpec((1, N_HEAD), lambda qi: (0, 0)),
            pl.BlockSpec((len(STRIDES), T // BK, BQ, BK), lambda qi: (0, 0, 0, 0)),
            pl.BlockSpec((N_EMBD, N_EMBD), lambda qi: (0, 0)),
            pl.BlockSpec((N_EMBD,), lambda qi: (0,)),
        ],
        out_specs=pl.BlockSpec((BQ, N_EMBD), lambda qi: (qi, 0)),
        out_shape=jax.ShapeDtypeStruct((T, N_EMBD), jnp.float32),
        interpret=interpret,
    )(qk, vv, kmax, tiles, W_out.astype(jnp.bfloat16), b_out)

    return out.reshape(1, T, N_EMBD)


def kernel(x, W_qk, b_qk, W_v, b_v, W_out, b_out):
    return _run(x, W_qk, b_qk, W_v, b_v, W_out, b_out)
